# baseline (speedup 1.0000x reference)
import functools

import jax
import jax.numpy as jnp
from jax import lax
from jax.experimental import pallas as pl
from jax.experimental.pallas import tpu as pltpu
from jax.experimental.pallas import tpu_sc as plsc

DIM = 16
NW = 32
BPW = 512
NBB = BPW // 128


def _gather_call(B, L):
    n_flat = B * L * DIM
    l_stride = B * DIM
    mesh = plsc.VectorSubcoreMesh(core_axis_name="c", subcore_axis_name="s")

    @functools.partial(
        pl.kernel,
        mesh=mesh,
        out_type=jax.ShapeDtypeStruct((n_flat,), jnp.float32),
        scratch_types=[
            pltpu.VMEM((2, BPW), jnp.int32),
            pltpu.VMEM((2, BPW, DIM), jnp.float32),
            pltpu.VMEM((2, 2 * NBB * 1024), jnp.float32),
            pltpu.SemaphoreType.DMA,
            pltpu.SemaphoreType.DMA,
            pltpu.SemaphoreType.DMA,
            pltpu.SemaphoreType.DMA,
        ],
        compiler_params=pltpu.CompilerParams(
            use_tc_tiling_on_sc=False, needs_layout_passes=False),
    )
    def emb(idx_hbm, table_hbm, out_hbm, idxbuf, rowbuf, tbuf,
            gsem0, gsem1, osem0, osem1):
        wid = lax.axis_index("s") * 2 + lax.axis_index("c")
        b0 = wid * BPW
        gsems = (gsem0, gsem1)
        osems = (osem0, osem1)
        iota = lax.iota(jnp.int32, 16)
        dpos = (iota // 8) * (NBB * 1024) + (iota % 8) * 128

        def load_and_fire(l, r):
            pltpu.sync_copy(idx_hbm.at[l, pl.ds(b0, BPW)], idxbuf.at[r])

            def add_body(i, c):
                for s in range(8):
                    sl = pl.ds(i * 128 + s * 16, 16)
                    idxbuf[r, sl] = idxbuf[r, sl] + 1
                return c

            lax.fori_loop(0, BPW // 128, add_body, 0)
            for j in range(NBB):
                pltpu.async_copy(
                    table_hbm.at[idxbuf.at[r].at[pl.ds(j * 128, 128)]],
                    rowbuf.at[r].at[pl.ds(j * 128, 128)],
                    gsems[r],
                )

        def drain_gathers(r):
            pltpu.make_async_copy(
                table_hbm.at[pl.ds(0, BPW)], rowbuf.at[r], gsems[r]
            ).wait()

        def transpose_and_store(l, r):
            def tr_body(k, c):
                bb = k // 8
                bg = k % 8
                base = dpos + (bb * 1024 + bg * 16)
                roff = bb * 128 + bg * 16
                for j in range(16):
                    row = rowbuf[r, roff + j, :]
                    plsc.store_scatter(tbuf.at[r], [base + j], row)
                return c

            lax.fori_loop(0, NBB * 8, tr_body, 0)
            off = l * l_stride + wid * (NBB * 1024)
            pltpu.async_copy(
                tbuf.at[r].at[pl.ds(0, NBB * 1024)],
                out_hbm.at[pl.ds(off, NBB * 1024)], osems[r])
            pltpu.async_copy(
                tbuf.at[r].at[pl.ds(NBB * 1024, NBB * 1024)],
                out_hbm.at[pl.ds(off + (B * DIM // 2), NBB * 1024)], osems[r])

        def wait_store(l, r):
            off = l * l_stride + wid * (NBB * 1024)
            pltpu.make_async_copy(
                tbuf.at[r], out_hbm.at[pl.ds(off, 2 * NBB * 1024)], osems[r]
            ).wait()

        load_and_fire(0, 0)

        def half_body(l, r):
            @pl.when(l < L - 1)
            def _():
                load_and_fire(l + 1, 1 - r)

            drain_gathers(r)

            @pl.when(l >= 2)
            def _():
                wait_store(l - 2, r)

            transpose_and_store(l, r)

        def pair_body(g, carry):
            half_body(2 * g, 0)
            half_body(2 * g + 1, 1)
            return carry

        lax.fori_loop(0, L // 2, pair_body, 0)
        wait_store(L - 2, 0)
        wait_store(L - 1, 1)

    return emb


def _retag_call(B, L):
    n_tiles = B * L * DIM // 1024
    tiles_per_w = n_tiles // NW
    tiles_per_l = B * DIM // 1024
    mesh = plsc.VectorSubcoreMesh(core_axis_name="c", subcore_axis_name="s")

    @functools.partial(
        pl.kernel,
        mesh=mesh,
        out_type=jax.ShapeDtypeStruct((L, DIM, B), jnp.float32),
        scratch_types=[
            pltpu.SemaphoreType.DMA,
        ],
        compiler_params=pltpu.CompilerParams(use_tc_tiling_on_sc=True),
    )
    def retag(in_hbm, out_hbm, sem):
        wid = lax.axis_index("s") * 2 + lax.axis_index("c")
        t0 = wid * tiles_per_w
        depth = 16

        def out_slice(t):
            l = t // tiles_per_l
            rem = t % tiles_per_l
            dh = rem // (tiles_per_l // 2)
            bb = rem % (tiles_per_l // 2)
            return out_hbm.at[l, pl.ds(dh * 8, 8), pl.ds(bb * 128, 128)]

        def body(i, c):
            t = t0 + i
            pltpu.async_copy(in_hbm.at[pl.ds(t * 8, 8)], out_slice(t), sem)

            @pl.when(i >= depth)
            def _():
                pltpu.make_async_copy(
                    in_hbm.at[pl.ds(t * 8, 8)], out_slice(t), sem).wait()
            return c

        lax.fori_loop(0, tiles_per_w, body, 0)
        for _ in range(depth):
            pltpu.make_async_copy(
                in_hbm.at[pl.ds(t0 * 8, 8)], out_slice(t0), sem).wait()

    return retag


def kernel(action, table):
    B, L, _ = action.shape
    idx2d = jnp.swapaxes(action.squeeze(-1), 0, 1)
    flat = _gather_call(B, L)(idx2d, table)
    tiled = _retag_call(B, L)(flat.reshape(B * L * DIM // 128, 128))
    return lax.transpose(tiled, (2, 0, 1))


# speedup vs baseline: 4.9158x; 4.9158x over previous
import functools

import jax
import jax.numpy as jnp
from jax import lax
from jax.experimental import pallas as pl
from jax.experimental.pallas import tpu as pltpu
from jax.experimental.pallas import tpu_sc as plsc

DIM = 16
NW = 32
BPW = 512
NBB = BPW // 128


def _gather_call(B, L):
    n_flat = B * L * DIM
    l_stride = B * DIM
    mesh = plsc.VectorSubcoreMesh(core_axis_name="c", subcore_axis_name="s")

    @functools.partial(
        pl.kernel,
        mesh=mesh,
        out_type=jax.ShapeDtypeStruct((n_flat,), jnp.float32),
        scratch_types=[
            pltpu.VMEM((2, BPW), jnp.int32),
            pltpu.VMEM((2, BPW, DIM), jnp.float32),
            pltpu.VMEM((2, 2 * NBB * 1024), jnp.float32),
            pltpu.SemaphoreType.DMA,
            pltpu.SemaphoreType.DMA,
            pltpu.SemaphoreType.DMA,
            pltpu.SemaphoreType.DMA,
        ],
        compiler_params=pltpu.CompilerParams(
            use_tc_tiling_on_sc=False, needs_layout_passes=False),
    )
    def emb(idx_hbm, table_hbm, out_hbm, idxbuf, rowbuf, tbuf,
            gsem0, gsem1, osem0, osem1):
        wid = lax.axis_index("s") * 2 + lax.axis_index("c")
        b0 = wid * BPW
        gsems = (gsem0, gsem1)
        osems = (osem0, osem1)
        iota = lax.iota(jnp.int32, 16)
        dpos = (iota // 8) * (NBB * 1024) + (iota % 8) * 128

        def load_and_fire(l, r):
            pltpu.sync_copy(idx_hbm.at[l, pl.ds(b0, BPW)], idxbuf.at[r])

            def add_body(i, c):
                for s in range(8):
                    sl = pl.ds(i * 128 + s * 16, 16)
                    idxbuf[r, sl] = idxbuf[r, sl] + 1
                return c

            lax.fori_loop(0, BPW // 128, add_body, 0)
            for j in range(NBB):
                pltpu.async_copy(
                    table_hbm.at[idxbuf.at[r].at[pl.ds(j * 128, 128)]],
                    rowbuf.at[r].at[pl.ds(j * 128, 128)],
                    gsems[r],
                )

        def drain_gathers(r):
            pltpu.make_async_copy(
                table_hbm.at[pl.ds(0, BPW)], rowbuf.at[r], gsems[r]
            ).wait()

        def transpose_and_store(l, r):
            def tr_body(k, c):
                bb = k // 8
                bg = k % 8
                rvec = bb * 128 + bg * 16 + iota
                for dh in range(2):
                    for dr in range(8):
                        d = dh * 8 + dr
                        lanes = plsc.load_gather(
                            rowbuf.at[r], [rvec, iota * 0 + d])
                        tbuf[r, pl.ds(dh * (NBB * 1024) + bb * 1024
                                      + dr * 128 + bg * 16, 16)] = lanes
                return c

            lax.fori_loop(0, NBB * 8, tr_body, 0)
            off = l * l_stride + wid * (NBB * 1024)
            pltpu.async_copy(
                tbuf.at[r].at[pl.ds(0, NBB * 1024)],
                out_hbm.at[pl.ds(off, NBB * 1024)], osems[r])
            pltpu.async_copy(
                tbuf.at[r].at[pl.ds(NBB * 1024, NBB * 1024)],
                out_hbm.at[pl.ds(off + (B * DIM // 2), NBB * 1024)], osems[r])

        def wait_store(l, r):
            off = l * l_stride + wid * (NBB * 1024)
            pltpu.make_async_copy(
                tbuf.at[r], out_hbm.at[pl.ds(off, 2 * NBB * 1024)], osems[r]
            ).wait()

        load_and_fire(0, 0)

        def half_body(l, r):
            @pl.when(l < L - 1)
            def _():
                load_and_fire(l + 1, 1 - r)

            drain_gathers(r)

            @pl.when(l >= 2)
            def _():
                wait_store(l - 2, r)

            transpose_and_store(l, r)

        def pair_body(g, carry):
            half_body(2 * g, 0)
            half_body(2 * g + 1, 1)
            return carry

        lax.fori_loop(0, L // 2, pair_body, 0)
        wait_store(L - 2, 0)
        wait_store(L - 1, 1)

    return emb


def _retag_call(B, L):
    n_tiles = B * L * DIM // 1024
    tiles_per_w = n_tiles // NW
    tiles_per_l = B * DIM // 1024
    mesh = plsc.VectorSubcoreMesh(core_axis_name="c", subcore_axis_name="s")

    @functools.partial(
        pl.kernel,
        mesh=mesh,
        out_type=jax.ShapeDtypeStruct((L, DIM, B), jnp.float32),
        scratch_types=[
            pltpu.VMEM((2, 8, 8, 128), jnp.float32),
            pltpu.SemaphoreType.DMA,
            pltpu.SemaphoreType.DMA,
            pltpu.SemaphoreType.DMA,
            pltpu.SemaphoreType.DMA,
        ],
        compiler_params=pltpu.CompilerParams(use_tc_tiling_on_sc=True),
    )
    def retag(in_hbm, out_hbm, buf, lsem0, lsem1, ssem0, ssem1):
        wid = lax.axis_index("s") * 2 + lax.axis_index("c")
        t0 = wid * tiles_per_w
        n_groups = tiles_per_w // 8
        lsems = (lsem0, lsem1)
        ssems = (ssem0, ssem1)

        def out_slice(t):
            l = t // tiles_per_l
            rem = t % tiles_per_l
            dh = rem // (tiles_per_l // 2)
            bb = rem % (tiles_per_l // 2)
            return out_hbm.at[l, pl.ds(dh * 8, 8), pl.ds(bb * 128, 128)]

        def group(g, s):
            tg = t0 + g * 8
            @pl.when(g >= 2)
            def _():
                for k in range(8):
                    pltpu.make_async_copy(
                        buf.at[s].at[k], out_slice(tg - 16 + k), ssems[s]
                    ).wait()

            for k in range(8):
                pltpu.async_copy(
                    in_hbm.at[pl.ds((tg + k) * 8, 8)], buf.at[s].at[k],
                    lsems[s])
            for k in range(8):
                pltpu.make_async_copy(
                    in_hbm.at[pl.ds((tg + k) * 8, 8)], buf.at[s].at[k],
                    lsems[s]).wait()
            for k in range(8):
                pltpu.async_copy(buf.at[s].at[k], out_slice(tg + k), ssems[s])

        def pair_body(q, carry):
            group(2 * q, 0)
            group(2 * q + 1, 1)
            return carry

        lax.fori_loop(0, n_groups // 2, pair_body, 0)
        for s in range(2):
            tg = t0 + (n_groups - 2 + s) * 8
            for k in range(8):
                pltpu.make_async_copy(
                    buf.at[s].at[k], out_slice(tg + k), ssems[s]).wait()

    return retag


def kernel(action, table):
    B, L, _ = action.shape
    idx2d = jnp.swapaxes(action.squeeze(-1), 0, 1)
    flat = _gather_call(B, L)(idx2d, table)
    tiled = _retag_call(B, L)(flat.reshape(B * L * DIM // 128, 128))
    return lax.transpose(tiled, (2, 0, 1))


# speedup vs baseline: 7.1055x; 1.4454x over previous
import functools

import jax
import jax.numpy as jnp
from jax import lax
from jax.experimental import pallas as pl
from jax.experimental.pallas import tpu as pltpu
from jax.experimental.pallas import tpu_sc as plsc

DIM = 16
NW = 32
BPW = 512
NBB = BPW // 128


def _gather_call(B, L):
    n_flat = B * L * DIM
    l_stride = B * DIM
    mesh = plsc.VectorSubcoreMesh(core_axis_name="c", subcore_axis_name="s")

    @functools.partial(
        pl.kernel,
        mesh=mesh,
        out_type=jax.ShapeDtypeStruct((n_flat,), jnp.float32),
        scratch_types=[
            pltpu.VMEM((2, BPW), jnp.int32),
            pltpu.VMEM((2, BPW, DIM), jnp.float32),
            pltpu.VMEM((2, 2 * NBB * 1024), jnp.float32),
            pltpu.SemaphoreType.DMA,
            pltpu.SemaphoreType.DMA,
            pltpu.SemaphoreType.DMA,
            pltpu.SemaphoreType.DMA,
        ],
        compiler_params=pltpu.CompilerParams(
            use_tc_tiling_on_sc=False, needs_layout_passes=False),
    )
    def emb(idx_hbm, table_hbm, out_hbm, idxbuf, rowbuf, tbuf,
            gsem0, gsem1, osem0, osem1):
        wid = lax.axis_index("s") * 2 + lax.axis_index("c")
        b0 = wid * BPW
        gsems = (gsem0, gsem1)
        osems = (osem0, osem1)
        iota = lax.iota(jnp.int32, 16)
        dpos = (iota // 8) * (NBB * 1024) + (iota % 8) * 128

        def load_and_fire(l, r):
            pltpu.sync_copy(idx_hbm.at[l, pl.ds(b0, BPW)], idxbuf.at[r])

            def add_body(i, c):
                for s in range(8):
                    sl = pl.ds(i * 128 + s * 16, 16)
                    idxbuf[r, sl] = idxbuf[r, sl] + 1
                return c

            lax.fori_loop(0, BPW // 128, add_body, 0)
            for j in range(NBB):
                pltpu.async_copy(
                    table_hbm.at[idxbuf.at[r].at[pl.ds(j * 128, 128)]],
                    rowbuf.at[r].at[pl.ds(j * 128, 128)],
                    gsems[r],
                )

        def drain_gathers(r):
            pltpu.make_async_copy(
                table_hbm.at[pl.ds(0, BPW)], rowbuf.at[r], gsems[r]
            ).wait()

        def transpose_and_store(l, r):
            def tr_body(k, c):
                bb = k // 8
                bg = k % 8
                rvec = bb * 128 + bg * 16 + iota
                lanes = [
                    plsc.load_gather(rowbuf.at[r], [rvec, iota * 0 + d])
                    for d in range(16)
                ]
                for dh in range(2):
                    for dr in range(8):
                        tbuf[r, pl.ds(dh * (NBB * 1024) + bb * 1024
                                      + dr * 128 + bg * 16, 16)] \
                            = lanes[dh * 8 + dr]
                return c

            lax.fori_loop(0, NBB * 8, tr_body, 0)
            off = l * l_stride + wid * (NBB * 1024)
            pltpu.async_copy(
                tbuf.at[r].at[pl.ds(0, NBB * 1024)],
                out_hbm.at[pl.ds(off, NBB * 1024)], osems[r])
            pltpu.async_copy(
                tbuf.at[r].at[pl.ds(NBB * 1024, NBB * 1024)],
                out_hbm.at[pl.ds(off + (B * DIM // 2), NBB * 1024)], osems[r])

        def wait_store(l, r):
            off = l * l_stride + wid * (NBB * 1024)
            pltpu.make_async_copy(
                tbuf.at[r], out_hbm.at[pl.ds(off, 2 * NBB * 1024)], osems[r]
            ).wait()

        load_and_fire(0, 0)

        def half_body(l, r):
            @pl.when(l < L - 1)
            def _():
                load_and_fire(l + 1, 1 - r)

            drain_gathers(r)

            @pl.when(l >= 2)
            def _():
                wait_store(l - 2, r)

            transpose_and_store(l, r)

        def pair_body(g, carry):
            half_body(2 * g, 0)
            half_body(2 * g + 1, 1)
            return carry

        lax.fori_loop(0, L // 2, pair_body, 0)
        wait_store(L - 2, 0)
        wait_store(L - 1, 1)

    return emb


def _retag_call(B, L):
    n_tiles = B * L * DIM // 1024
    tiles_per_w = n_tiles // NW
    tiles_per_l = B * DIM // 1024
    mesh = plsc.VectorSubcoreMesh(core_axis_name="c", subcore_axis_name="s")

    @functools.partial(
        pl.kernel,
        mesh=mesh,
        out_type=jax.ShapeDtypeStruct((L, DIM, B), jnp.float32),
        scratch_types=[
            pltpu.VMEM((2, 8, 8, 128), jnp.float32),
            pltpu.SemaphoreType.DMA,
            pltpu.SemaphoreType.DMA,
            pltpu.SemaphoreType.DMA,
            pltpu.SemaphoreType.DMA,
        ],
        compiler_params=pltpu.CompilerParams(use_tc_tiling_on_sc=True),
    )
    def retag(in_hbm, out_hbm, buf, lsem0, lsem1, ssem0, ssem1):
        wid = lax.axis_index("s") * 2 + lax.axis_index("c")
        t0 = wid * tiles_per_w
        n_groups = tiles_per_w // 8
        lsems = (lsem0, lsem1)
        ssems = (ssem0, ssem1)

        def out_slice(t):
            l = t // tiles_per_l
            rem = t % tiles_per_l
            dh = rem // (tiles_per_l // 2)
            bb = rem % (tiles_per_l // 2)
            return out_hbm.at[l, pl.ds(dh * 8, 8), pl.ds(bb * 128, 128)]

        def group(g, s):
            tg = t0 + g * 8
            @pl.when(g >= 2)
            def _():
                for k in range(8):
                    pltpu.make_async_copy(
                        buf.at[s].at[k], out_slice(tg - 16 + k), ssems[s]
                    ).wait()

            for k in range(8):
                pltpu.async_copy(
                    in_hbm.at[pl.ds((tg + k) * 8, 8)], buf.at[s].at[k],
                    lsems[s])
            for k in range(8):
                pltpu.make_async_copy(
                    in_hbm.at[pl.ds((tg + k) * 8, 8)], buf.at[s].at[k],
                    lsems[s]).wait()
            for k in range(8):
                pltpu.async_copy(buf.at[s].at[k], out_slice(tg + k), ssems[s])

        def pair_body(q, carry):
            group(2 * q, 0)
            group(2 * q + 1, 1)
            return carry

        lax.fori_loop(0, n_groups // 2, pair_body, 0)
        for s in range(2):
            tg = t0 + (n_groups - 2 + s) * 8
            for k in range(8):
                pltpu.make_async_copy(
                    buf.at[s].at[k], out_slice(tg + k), ssems[s]).wait()

    return retag


def kernel(action, table):
    B, L, _ = action.shape
    idx2d = jnp.swapaxes(action.squeeze(-1), 0, 1)
    flat = _gather_call(B, L)(idx2d, table)
    tiled = _retag_call(B, L)(flat.reshape(B * L * DIM // 128, 128))
    return lax.transpose(tiled, (2, 0, 1))


# speedup vs baseline: 7.5235x; 1.0588x over previous
import functools

import jax
import jax.numpy as jnp
from jax import lax
from jax.experimental import pallas as pl
from jax.experimental.pallas import tpu as pltpu
from jax.experimental.pallas import tpu_sc as plsc

DIM = 16
NW = 32
BPW = 512
NBB = BPW // 128


def _gather_call(B, L):
    n_flat = B * L * DIM
    l_stride = B * DIM
    mesh = plsc.VectorSubcoreMesh(core_axis_name="c", subcore_axis_name="s")

    @functools.partial(
        pl.kernel,
        mesh=mesh,
        out_type=jax.ShapeDtypeStruct((n_flat,), jnp.float32),
        scratch_types=[
            pltpu.VMEM((2, BPW), jnp.int32),
            pltpu.VMEM((2, BPW, DIM), jnp.float32),
            pltpu.VMEM((2, 2 * NBB * 1024), jnp.float32),
            pltpu.SemaphoreType.DMA,
            pltpu.SemaphoreType.DMA,
            pltpu.SemaphoreType.DMA,
            pltpu.SemaphoreType.DMA,
        ],
        compiler_params=pltpu.CompilerParams(
            use_tc_tiling_on_sc=False, needs_layout_passes=False),
    )
    def emb(idx_hbm, table_hbm, out_hbm, idxbuf, rowbuf, tbuf,
            gsem0, gsem1, osem0, osem1):
        wid = lax.axis_index("s") * 2 + lax.axis_index("c")
        b0 = wid * BPW
        gsems = (gsem0, gsem1)
        osems = (osem0, osem1)
        iota = lax.iota(jnp.int32, 16)
        dpos = (iota // 8) * (NBB * 1024) + (iota % 8) * 128

        def load_and_fire(l, r):
            pltpu.sync_copy(idx_hbm.at[l, pl.ds(b0, BPW)], idxbuf.at[r])

            def add_body(i, c):
                for s in range(8):
                    sl = pl.ds(i * 128 + s * 16, 16)
                    idxbuf[r, sl] = idxbuf[r, sl] + 1
                return c

            lax.fori_loop(0, BPW // 128, add_body, 0)
            for j in range(NBB):
                pltpu.async_copy(
                    table_hbm.at[idxbuf.at[r].at[pl.ds(j * 128, 128)]],
                    rowbuf.at[r].at[pl.ds(j * 128, 128)],
                    gsems[r],
                )

        def drain_gathers(r):
            pltpu.make_async_copy(
                table_hbm.at[pl.ds(0, BPW)], rowbuf.at[r], gsems[r]
            ).wait()

        def transpose_and_store(l, r):
            def tr_body(k, c):
                bb = k // 8
                bg = k % 8
                rvec = bb * 128 + bg * 16 + iota
                lanes = [
                    plsc.load_gather(rowbuf.at[r], [rvec, iota * 0 + d])
                    for d in range(16)
                ]
                for dh in range(2):
                    for dr in range(8):
                        tbuf[r, pl.ds(dh * (NBB * 1024) + bb * 1024
                                      + dr * 128 + bg * 16, 16)] \
                            = lanes[dh * 8 + dr]
                return c

            lax.fori_loop(0, NBB * 8, tr_body, 0)
            off = l * l_stride + wid * (NBB * 1024)
            pltpu.async_copy(
                tbuf.at[r].at[pl.ds(0, NBB * 1024)],
                out_hbm.at[pl.ds(off, NBB * 1024)], osems[r])
            pltpu.async_copy(
                tbuf.at[r].at[pl.ds(NBB * 1024, NBB * 1024)],
                out_hbm.at[pl.ds(off + (B * DIM // 2), NBB * 1024)], osems[r])

        def wait_store(l, r):
            off = l * l_stride + wid * (NBB * 1024)
            pltpu.make_async_copy(
                tbuf.at[r], out_hbm.at[pl.ds(off, 2 * NBB * 1024)], osems[r]
            ).wait()

        load_and_fire(0, 0)

        def half_body(l, r):
            @pl.when(l < L - 1)
            def _():
                load_and_fire(l + 1, 1 - r)

            drain_gathers(r)

            @pl.when(l >= 2)
            def _():
                wait_store(l - 2, r)

            transpose_and_store(l, r)

        def pair_body(g, carry):
            half_body(2 * g, 0)
            half_body(2 * g + 1, 1)
            return carry

        lax.fori_loop(0, L // 2, pair_body, 0)
        wait_store(L - 2, 0)
        wait_store(L - 1, 1)

    return emb


def _retag_call(B, L):
    n_tiles = B * L * DIM // 1024
    tiles_per_w = n_tiles // NW
    tiles_per_l = B * DIM // 1024
    mesh = plsc.VectorSubcoreMesh(core_axis_name="c", subcore_axis_name="s")

    @functools.partial(
        pl.kernel,
        mesh=mesh,
        out_type=jax.ShapeDtypeStruct((L, DIM, B), jnp.float32),
        scratch_types=[
            pltpu.VMEM((2, 16, 8, 128), jnp.float32),
            pltpu.SemaphoreType.DMA,
            pltpu.SemaphoreType.DMA,
            pltpu.SemaphoreType.DMA,
            pltpu.SemaphoreType.DMA,
        ],
        compiler_params=pltpu.CompilerParams(use_tc_tiling_on_sc=True),
    )
    def retag(in_hbm, out_hbm, buf, lsem0, lsem1, ssem0, ssem1):
        wid = lax.axis_index("s") * 2 + lax.axis_index("c")
        t0 = wid * tiles_per_w
        n_groups = tiles_per_w // 16
        lsems = (lsem0, lsem1)
        ssems = (ssem0, ssem1)

        def out_slice(t):
            l = t // tiles_per_l
            rem = t % tiles_per_l
            dh = rem // (tiles_per_l // 2)
            bb = rem % (tiles_per_l // 2)
            return out_hbm.at[l, pl.ds(dh * 8, 8), pl.ds(bb * 128, 128)]

        def group(g, s):
            tg = t0 + g * 16
            @pl.when(g >= 2)
            def _():
                for k in range(16):
                    pltpu.make_async_copy(
                        buf.at[s].at[k], out_slice(tg - 32 + k), ssems[s]
                    ).wait()

            for k in range(16):
                pltpu.async_copy(
                    in_hbm.at[pl.ds((tg + k) * 8, 8)], buf.at[s].at[k],
                    lsems[s])
            for k in range(16):
                pltpu.make_async_copy(
                    in_hbm.at[pl.ds((tg + k) * 8, 8)], buf.at[s].at[k],
                    lsems[s]).wait()
            for k in range(16):
                pltpu.async_copy(buf.at[s].at[k], out_slice(tg + k), ssems[s])

        def pair_body(q, carry):
            group(2 * q, 0)
            group(2 * q + 1, 1)
            return carry

        lax.fori_loop(0, n_groups // 2, pair_body, 0)
        for s in range(2):
            tg = t0 + (n_groups - 2 + s) * 16
            for k in range(16):
                pltpu.make_async_copy(
                    buf.at[s].at[k], out_slice(tg + k), ssems[s]).wait()

    return retag


def kernel(action, table):
    B, L, _ = action.shape
    idx2d = jnp.swapaxes(action.squeeze(-1), 0, 1)
    flat = _gather_call(B, L)(idx2d, table)
    tiled = _retag_call(B, L)(flat.reshape(B * L * DIM // 128, 128))
    return lax.transpose(tiled, (2, 0, 1))


# speedup vs baseline: 7.7800x; 1.0341x over previous
import functools

import jax
import jax.numpy as jnp
from jax import lax
from jax.experimental import pallas as pl
from jax.experimental.pallas import tpu as pltpu
from jax.experimental.pallas import tpu_sc as plsc

DIM = 16
NW = 32
BPW = 512
NBB = BPW // 128


def _gather_call(B, L):
    n_flat = B * L * DIM
    l_stride = B * DIM
    mesh = plsc.VectorSubcoreMesh(core_axis_name="c", subcore_axis_name="s")

    @functools.partial(
        pl.kernel,
        mesh=mesh,
        out_type=jax.ShapeDtypeStruct((n_flat,), jnp.float32),
        scratch_types=[
            pltpu.VMEM((2, BPW), jnp.int32),
            pltpu.VMEM((2, BPW, DIM), jnp.float32),
            pltpu.VMEM((2, 2 * NBB * 1024), jnp.float32),
            pltpu.SemaphoreType.DMA,
            pltpu.SemaphoreType.DMA,
            pltpu.SemaphoreType.DMA,
            pltpu.SemaphoreType.DMA,
        ],
        compiler_params=pltpu.CompilerParams(
            use_tc_tiling_on_sc=False, needs_layout_passes=False),
    )
    def emb(idx_hbm, table_hbm, out_hbm, idxbuf, rowbuf, tbuf,
            gsem0, gsem1, osem0, osem1):
        wid = lax.axis_index("s") * 2 + lax.axis_index("c")
        b0 = wid * BPW
        gsems = (gsem0, gsem1)
        osems = (osem0, osem1)
        iota = lax.iota(jnp.int32, 16)
        dpos = (iota // 8) * (NBB * 1024) + (iota % 8) * 128

        def load_and_fire(l, r):
            pltpu.sync_copy(idx_hbm.at[l, pl.ds(b0, BPW)], idxbuf.at[r])

            def add_body(i, c):
                for s in range(8):
                    sl = pl.ds(i * 128 + s * 16, 16)
                    idxbuf[r, sl] = idxbuf[r, sl] + 1
                return c

            lax.fori_loop(0, BPW // 128, add_body, 0)
            for j in range(NBB):
                pltpu.async_copy(
                    table_hbm.at[idxbuf.at[r].at[pl.ds(j * 128, 128)]],
                    rowbuf.at[r].at[pl.ds(j * 128, 128)],
                    gsems[r],
                )

        def drain_gathers(r):
            pltpu.make_async_copy(
                table_hbm.at[pl.ds(0, BPW)], rowbuf.at[r], gsems[r]
            ).wait()

        def transpose_and_store(l, r):
            def tr_body(k, c):
                bb = k // 8
                bg = k % 8
                rvec = bb * 128 + bg * 16 + iota
                lanes = [
                    plsc.load_gather(rowbuf.at[r], [rvec, iota * 0 + d])
                    for d in range(16)
                ]
                for dh in range(2):
                    for dr in range(8):
                        tbuf[r, pl.ds(dh * (NBB * 1024) + bb * 1024
                                      + dr * 128 + bg * 16, 16)] \
                            = lanes[dh * 8 + dr]
                return c

            lax.fori_loop(0, NBB * 8, tr_body, 0)
            off = l * l_stride + wid * (NBB * 1024)
            pltpu.async_copy(
                tbuf.at[r].at[pl.ds(0, NBB * 1024)],
                out_hbm.at[pl.ds(off, NBB * 1024)], osems[r])
            pltpu.async_copy(
                tbuf.at[r].at[pl.ds(NBB * 1024, NBB * 1024)],
                out_hbm.at[pl.ds(off + (B * DIM // 2), NBB * 1024)], osems[r])

        def wait_store(l, r):
            off = l * l_stride + wid * (NBB * 1024)
            pltpu.make_async_copy(
                tbuf.at[r], out_hbm.at[pl.ds(off, 2 * NBB * 1024)], osems[r]
            ).wait()

        load_and_fire(0, 0)

        def half_body(l, r):
            @pl.when(l < L - 1)
            def _():
                load_and_fire(l + 1, 1 - r)

            drain_gathers(r)

            @pl.when(l >= 2)
            def _():
                wait_store(l - 2, r)

            transpose_and_store(l, r)

        def pair_body(g, carry):
            half_body(2 * g, 0)
            half_body(2 * g + 1, 1)
            return carry

        lax.fori_loop(0, L // 2, pair_body, 0)
        wait_store(L - 2, 0)
        wait_store(L - 1, 1)

    return emb


def _retag_call(B, L):
    n_tiles = B * L * DIM // 1024
    tiles_per_w = n_tiles // NW
    tiles_per_l = B * DIM // 1024
    mesh = plsc.VectorSubcoreMesh(core_axis_name="c", subcore_axis_name="s")

    @functools.partial(
        pl.kernel,
        mesh=mesh,
        out_type=jax.ShapeDtypeStruct((L, DIM, B), jnp.float32),
        scratch_types=[
            pltpu.VMEM((2, 32, 8, 128), jnp.float32),
            pltpu.SemaphoreType.DMA,
            pltpu.SemaphoreType.DMA,
            pltpu.SemaphoreType.DMA,
            pltpu.SemaphoreType.DMA,
        ],
        compiler_params=pltpu.CompilerParams(use_tc_tiling_on_sc=True),
    )
    def retag(in_hbm, out_hbm, buf, lsem0, lsem1, ssem0, ssem1):
        wid = lax.axis_index("s") * 2 + lax.axis_index("c")
        t0 = wid * tiles_per_w
        n_groups = tiles_per_w // 32
        lsems = (lsem0, lsem1)
        ssems = (ssem0, ssem1)

        def out_slice(t):
            l = t // tiles_per_l
            rem = t % tiles_per_l
            dh = rem // (tiles_per_l // 2)
            bb = rem % (tiles_per_l // 2)
            return out_hbm.at[l, pl.ds(dh * 8, 8), pl.ds(bb * 128, 128)]

        def group(g, s):
            tg = t0 + g * 32
            @pl.when(g >= 2)
            def _():
                for k in range(32):
                    pltpu.make_async_copy(
                        buf.at[s].at[k], out_slice(tg - 64 + k), ssems[s]
                    ).wait()

            for k in range(32):
                pltpu.async_copy(
                    in_hbm.at[pl.ds((tg + k) * 8, 8)], buf.at[s].at[k],
                    lsems[s])
            for k in range(32):
                pltpu.make_async_copy(
                    in_hbm.at[pl.ds((tg + k) * 8, 8)], buf.at[s].at[k],
                    lsems[s]).wait()
            for k in range(32):
                pltpu.async_copy(buf.at[s].at[k], out_slice(tg + k), ssems[s])

        def pair_body(q, carry):
            group(2 * q, 0)
            group(2 * q + 1, 1)
            return carry

        lax.fori_loop(0, n_groups // 2, pair_body, 0)
        for s in range(2):
            tg = t0 + (n_groups - 2 + s) * 32
            for k in range(32):
                pltpu.make_async_copy(
                    buf.at[s].at[k], out_slice(tg + k), ssems[s]).wait()

    return retag


def kernel(action, table):
    B, L, _ = action.shape
    idx2d = jnp.swapaxes(action.squeeze(-1), 0, 1)
    flat = _gather_call(B, L)(idx2d, table)
    tiled = _retag_call(B, L)(flat.reshape(B * L * DIM // 128, 128))
    return lax.transpose(tiled, (2, 0, 1))


# speedup vs baseline: 7.7828x; 1.0004x over previous
import functools

import jax
import jax.numpy as jnp
from jax import lax
from jax.experimental import pallas as pl
from jax.experimental.pallas import tpu as pltpu
from jax.experimental.pallas import tpu_sc as plsc

DIM = 16
NW = 32
BPW = 512
NBB = BPW // 128


def _gather_call(B, L):
    n_flat = B * L * DIM
    l_stride = B * DIM
    mesh = plsc.VectorSubcoreMesh(core_axis_name="c", subcore_axis_name="s")

    @functools.partial(
        pl.kernel,
        mesh=mesh,
        out_type=jax.ShapeDtypeStruct((n_flat,), jnp.float32),
        scratch_types=[
            pltpu.VMEM((2, BPW), jnp.int32),
            pltpu.VMEM((2, BPW, DIM), jnp.float32),
            pltpu.VMEM((2, 2 * NBB * 1024), jnp.float32),
            pltpu.SemaphoreType.DMA,
            pltpu.SemaphoreType.DMA,
            pltpu.SemaphoreType.DMA,
            pltpu.SemaphoreType.DMA,
        ],
        compiler_params=pltpu.CompilerParams(
            use_tc_tiling_on_sc=False, needs_layout_passes=False),
    )
    def emb(idx_hbm, table_hbm, out_hbm, idxbuf, rowbuf, tbuf,
            gsem0, gsem1, osem0, osem1):
        wid = lax.axis_index("s") * 2 + lax.axis_index("c")
        b0 = wid * BPW
        gsems = (gsem0, gsem1)
        osems = (osem0, osem1)
        iota = lax.iota(jnp.int32, 16)

        def load_and_fire(l, r):
            pltpu.sync_copy(idx_hbm.at[l, pl.ds(b0, BPW)], idxbuf.at[r])

            def add_body(i, c):
                for s in range(8):
                    sl = pl.ds(i * 128 + s * 16, 16)
                    idxbuf[r, sl] = idxbuf[r, sl] + 1
                return c

            lax.fori_loop(0, BPW // 128, add_body, 0)
            for j in range(NBB):
                pltpu.async_copy(
                    table_hbm.at[idxbuf.at[r].at[pl.ds(j * 128, 128)]],
                    rowbuf.at[r].at[pl.ds(j * 128, 128)],
                    gsems[r],
                )

        def drain_gathers(r):
            pltpu.make_async_copy(
                table_hbm.at[pl.ds(0, BPW)], rowbuf.at[r], gsems[r]
            ).wait()

        def transpose_and_store(l, r):
            def tr_body(k, c):
                bb = k // 8
                bg = k % 8
                rvec = bb * 128 + bg * 16 + iota
                lanes = [
                    plsc.load_gather(rowbuf.at[r], [rvec, iota * 0 + d])
                    for d in range(16)
                ]
                for dh in range(2):
                    for dr in range(8):
                        tbuf[r, pl.ds(dh * (NBB * 1024) + bb * 1024
                                      + dr * 128 + bg * 16, 16)] \
                            = lanes[dh * 8 + dr]
                return c

            lax.fori_loop(0, NBB * 8, tr_body, 0)
            off = l * l_stride + wid * (NBB * 1024)
            pltpu.async_copy(
                tbuf.at[r].at[pl.ds(0, NBB * 1024)],
                out_hbm.at[pl.ds(off, NBB * 1024)], osems[r])
            pltpu.async_copy(
                tbuf.at[r].at[pl.ds(NBB * 1024, NBB * 1024)],
                out_hbm.at[pl.ds(off + (B * DIM // 2), NBB * 1024)], osems[r])

        def wait_store(l, r):
            off = l * l_stride + wid * (NBB * 1024)
            pltpu.make_async_copy(
                tbuf.at[r], out_hbm.at[pl.ds(off, 2 * NBB * 1024)], osems[r]
            ).wait()

        load_and_fire(0, 0)

        def half_body(l, r):
            @pl.when(l < L - 1)
            def _():
                load_and_fire(l + 1, 1 - r)

            drain_gathers(r)

            @pl.when(l >= 2)
            def _():
                wait_store(l - 2, r)

            transpose_and_store(l, r)

        def pair_body(g, carry):
            half_body(2 * g, 0)
            half_body(2 * g + 1, 1)
            return carry

        lax.fori_loop(0, L // 2, pair_body, 0)
        wait_store(L - 2, 0)
        wait_store(L - 1, 1)

    return emb


def _retag_call(B, L):
    n_tiles = B * L * DIM // 1024
    tiles_per_w = n_tiles // NW
    tiles_per_l = B * DIM // 1024
    mesh = plsc.VectorSubcoreMesh(core_axis_name="c", subcore_axis_name="s")

    @functools.partial(
        pl.kernel,
        mesh=mesh,
        out_type=jax.ShapeDtypeStruct((L, DIM, B), jnp.float32),
        scratch_types=[
            pltpu.VMEM((2, 32, 8, 128), jnp.float32),
            pltpu.SemaphoreType.DMA,
            pltpu.SemaphoreType.DMA,
            pltpu.SemaphoreType.DMA,
            pltpu.SemaphoreType.DMA,
        ],
        compiler_params=pltpu.CompilerParams(use_tc_tiling_on_sc=True),
    )
    def retag(in_hbm, out_hbm, buf, lsem0, lsem1, ssem0, ssem1):
        wid = lax.axis_index("s") * 2 + lax.axis_index("c")
        t0 = wid * tiles_per_w
        n_groups = tiles_per_w // 32
        lsems = (lsem0, lsem1)
        ssems = (ssem0, ssem1)

        def out_slice(t):
            l = t // tiles_per_l
            rem = t % tiles_per_l
            dh = rem // (tiles_per_l // 2)
            bb = rem % (tiles_per_l // 2)
            return out_hbm.at[l, pl.ds(dh * 8, 8), pl.ds(bb * 128, 128)]

        def group(g, s):
            tg = t0 + g * 32
            @pl.when(g >= 2)
            def _():
                for k in range(32):
                    pltpu.make_async_copy(
                        buf.at[s].at[k], out_slice(tg - 64 + k), ssems[s]
                    ).wait()

            for k in range(32):
                pltpu.async_copy(
                    in_hbm.at[pl.ds((tg + k) * 8, 8)], buf.at[s].at[k],
                    lsems[s])
            for k in range(32):
                pltpu.make_async_copy(
                    in_hbm.at[pl.ds((tg + k) * 8, 8)], buf.at[s].at[k],
                    lsems[s]).wait()
            for k in range(32):
                pltpu.async_copy(buf.at[s].at[k], out_slice(tg + k), ssems[s])

        def pair_body(q, carry):
            group(2 * q, 0)
            group(2 * q + 1, 1)
            return carry

        lax.fori_loop(0, n_groups // 2, pair_body, 0)
        for s in range(2):
            tg = t0 + (n_groups - 2 + s) * 32
            for k in range(32):
                pltpu.make_async_copy(
                    buf.at[s].at[k], out_slice(tg + k), ssems[s]).wait()

    return retag


def kernel(action, table):
    B, L, _ = action.shape
    idx2d = jnp.swapaxes(action.squeeze(-1), 0, 1)
    flat = _gather_call(B, L)(idx2d, table)
    tiled = _retag_call(B, L)(flat.reshape(B * L * DIM // 128, 128))
    return lax.transpose(tiled, (2, 0, 1))
